# Initial kernel scaffold; baseline (speedup 1.0000x reference)
import jax
import jax.numpy as jnp
from jax.experimental import pallas as pl


def kernel(x, edge_index, edge_type, rel_emb, W, W_self):
    raise NotImplementedError("write your pallas kernel here")


# speedup vs baseline: 10.9876x; 10.9876x over previous
import functools

import jax
import jax.numpy as jnp
from jax import lax
from jax.experimental import pallas as pl
from jax.experimental.pallas import tpu as pltpu
from jax.experimental.pallas import tpu_sc as plsc

N = 10000
E = 320000
D = 128
R = 8

NC = 2
NS = 16
NW = NC * NS
EPW = E // NW
CHUNK = 125
NCHUNK = EPW // CHUNK
RPT = N // NS
ZREP = RPT // CHUNK


def _table_body(x_ref, rel_ref, out_ref):
    r = pl.program_id(0)
    out_ref[...] = x_ref[...] * rel_ref[r, :][None, :]


def _build_table(x, rel_emb):
    return pl.pallas_call(
        _table_body,
        grid=(R,),
        in_specs=[
            pl.BlockSpec((N, D), lambda r: (0, 0)),
            pl.BlockSpec((R, D), lambda r: (0, 0)),
        ],
        out_specs=pl.BlockSpec((N, D), lambda r: (r, 0)),
        out_shape=jax.ShapeDtypeStruct((R * N, D), jnp.float32),
    )(x, rel_emb)


def _ci_body(src_ref, et_ref, out_ref):
    out_ref[...] = et_ref[...] * N + src_ref[...]


def _build_ci(src2, et2):
    return pl.pallas_call(
        _ci_body,
        out_shape=jax.ShapeDtypeStruct(src2.shape, jnp.int32),
    )(src2, et2)


def _sc_agg_body(table_hbm, ci_hbm, dst_hbm, out_hbm, ci_v, dst_v, rows_v,
                 agg_sh, sem):
    cid = lax.axis_index("c")
    sid = lax.axis_index("s")
    wid = sid * NC + cid

    pltpu.sync_copy(ci_hbm.at[wid], ci_v)
    pltpu.sync_copy(dst_hbm.at[wid], dst_v)

    def _zrow(r, c):
        for dd in range(D // 16):
            rows_v[r, pl.ds(dd * 16, 16)] = jnp.zeros((16,), jnp.float32)
        return c

    lax.fori_loop(0, CHUNK, _zrow, 0)
    for j in range(ZREP):
        pltpu.sync_copy(rows_v, agg_sh.at[pl.ds(sid * RPT + j * CHUNK, CHUNK)])
    plsc.subcore_barrier()

    def _edge_chunk(k, c):
        pltpu.async_copy(table_hbm.at[ci_v.at[k]], rows_v, sem).wait()
        pltpu.sync_copy(rows_v, agg_sh.at[dst_v.at[k]], add=True)
        return c

    lax.fori_loop(0, NCHUNK, _edge_chunk, 0)
    plsc.subcore_barrier()

    pltpu.sync_copy(agg_sh.at[pl.ds(sid * RPT, RPT)], out_hbm.at[cid, sid])


_sc_agg = functools.partial(
    pl.kernel,
    out_type=jax.ShapeDtypeStruct((NC, NS, RPT, D), jnp.float32),
    mesh=plsc.VectorSubcoreMesh(core_axis_name="c", subcore_axis_name="s"),
    scratch_types=[
        pltpu.VMEM((NCHUNK, CHUNK), jnp.int32),
        pltpu.VMEM((NCHUNK, CHUNK), jnp.int32),
        pltpu.VMEM((CHUNK, D), jnp.float32),
        pltpu.VMEM_SHARED((N, D), jnp.float32),
        pltpu.SemaphoreType.DMA,
    ],
)(_sc_agg_body)


def _out_body(agg_ref, x_ref, w_ref, ws_ref, out_ref):
    a = agg_ref[0] + agg_ref[1]
    out_ref[...] = (
        jnp.dot(a, w_ref[...], preferred_element_type=jnp.float32)
        + jnp.dot(x_ref[...], ws_ref[...], preferred_element_type=jnp.float32)
    )


def _combine(agg2, x, W, W_self):
    NB = 2000
    return pl.pallas_call(
        _out_body,
        grid=(N // NB,),
        in_specs=[
            pl.BlockSpec((NC, NB, D), lambda i: (0, i, 0)),
            pl.BlockSpec((NB, D), lambda i: (i, 0)),
            pl.BlockSpec((D, D), lambda i: (0, 0)),
            pl.BlockSpec((D, D), lambda i: (0, 0)),
        ],
        out_specs=pl.BlockSpec((NB, D), lambda i: (i, 0)),
        out_shape=jax.ShapeDtypeStruct((N, D), jnp.float32),
    )(agg2, x, W, W_self)


def kernel(x, edge_index, edge_type, rel_emb, W, W_self):
    src = edge_index[0].astype(jnp.int32)
    dst = edge_index[1].astype(jnp.int32)
    et = edge_type.astype(jnp.int32)

    table = _build_table(x, rel_emb)
    ci2 = _build_ci(src.reshape(E // D, D), et.reshape(E // D, D))

    ci3 = ci2.reshape(NW, NCHUNK, CHUNK)
    dst3 = dst.reshape(NW, NCHUNK, CHUNK)
    agg2 = _sc_agg(table, ci3, dst3).reshape(NC, N, D)
    return _combine(agg2, x, W, W_self)
